# baseline (device time: 11172 ns/iter reference)
import jax
import jax.numpy as jnp
from jax import lax
from jax.experimental import pallas as pl
from jax.experimental.pallas import tpu as pltpu

NZ = 4
M = 256
NCOL = 1024
CHUNK = NCOL // NZ

TARGET_ORDER = (0, 3, 1, 2)


def kernel(x):
    def body(x_ref, out_ref, send_buf, recv_buf, send_sems, recv_sems, arr_sems):
        my_x = lax.axis_index("x")
        my_y = lax.axis_index("y")
        my_z = lax.axis_index("z")

        barrier_sem = pltpu.get_barrier_semaphore()
        pl.semaphore_signal(barrier_sem, inc=1)
        pl.semaphore_wait(barrier_sem, 1)

        for t in TARGET_ORDER:
            pl.semaphore_signal(
                arr_sems.at[my_z],
                inc=1,
                device_id=(my_x, my_y, t),
                device_id_type=pl.DeviceIdType.MESH,
            )

        rdmas = []
        for t in TARGET_ORDER:
            send_buf[t] = x_ref[0, :, t * CHUNK : (t + 1) * CHUNK].astype(
                jnp.bfloat16
            )
            pl.semaphore_wait(arr_sems.at[t], 1)
            rdma = pltpu.make_async_remote_copy(
                src_ref=send_buf.at[t],
                dst_ref=recv_buf.at[my_z],
                send_sem=send_sems.at[t],
                recv_sem=recv_sems.at[my_z],
                device_id=(my_x, my_y, t),
                device_id_type=pl.DeviceIdType.MESH,
            )
            rdma.start()
            rdmas.append(rdma)

        recvs = []
        for s in range(NZ):
            recvs.append(
                pltpu.make_async_remote_copy(
                    src_ref=send_buf.at[s],
                    dst_ref=recv_buf.at[s],
                    send_sem=send_sems.at[s],
                    recv_sem=recv_sems.at[s],
                    device_id=(my_x, my_y, s),
                    device_id_type=pl.DeviceIdType.MESH,
                )
            )
        acc = None
        for s in range(NZ):
            recvs[s].wait_recv()
            term = recv_buf[s].astype(jnp.float32)
            acc = term if acc is None else acc + term
        out_ref[:, :] = acc

        for rdma in rdmas:
            rdma.wait_send()

    return pl.pallas_call(
        body,
        out_shape=jax.ShapeDtypeStruct((M, CHUNK), jnp.float32),
        in_specs=[pl.BlockSpec(memory_space=pltpu.VMEM)],
        out_specs=pl.BlockSpec(memory_space=pltpu.VMEM),
        scratch_shapes=[
            pltpu.VMEM((NZ, M, CHUNK), jnp.bfloat16),
            pltpu.VMEM((NZ, M, CHUNK), jnp.bfloat16),
            pltpu.SemaphoreType.DMA((NZ,)),
            pltpu.SemaphoreType.DMA((NZ,)),
            pltpu.SemaphoreType.REGULAR((NZ,)),
        ],
        compiler_params=pltpu.CompilerParams(collective_id=0),
    )(x)
